# baseline (speedup 1.0000x reference)
import jax
import jax.numpy as jnp
from jax import lax
from jax.experimental import pallas as pl
from jax.experimental.pallas import tpu as pltpu
from jax.experimental.pallas import tpu_sc as plsc

N = 10000
NPAD = 10240
NW = 32
NPW = NPAD // NW
E_TOT = 330000
S = 1024
EC = 11264
E_PAD = NW * EC
KD = 512
NL = 264
GQ = 64
SB_LEN = 64
NC, NS = 2, 16
NEG = float("-inf")
F32 = jnp.float32


def _mesh():
    return plsc.VectorSubcoreMesh(core_axis_name="c", subcore_axis_name="s")


def _wid():
    return lax.axis_index("s") * NC + lax.axis_index("c")


def _gather_body(t_hbm, t2_hbm, src_hbm, dst_hbm, u_hbm, si_v, di_v,
                 sb0, sb1, sb2, sb3, db0, db1, db2, db3,
                 ss0, ss1, ss2, ss3, sd0, sd1, sd2, sd3):
    base = pl.multiple_of(_wid() * EC, 8)
    pltpu.sync_copy(src_hbm.at[pl.ds(base, EC)], si_v)
    pltpu.sync_copy(dst_hbm.at[pl.ds(base, EC)], di_v)
    sbufs = (sb0, sb1, sb2, sb3)
    dbufs = (db0, db1, db2, db3)
    ssems = (ss0, ss1, ss2, ss3)
    dsems = (sd0, sd1, sd2, sd3)
    nstep = EC // GQ

    def fire(k, b):
        off = pl.multiple_of(k * GQ, 8)
        pltpu.async_copy(t_hbm.at[si_v.at[pl.ds(off, GQ)]], sbufs[b], ssems[b])
        pltpu.async_copy(t2_hbm.at[di_v.at[pl.ds(off, GQ)]], dbufs[b], dsems[b])

    def drain_s(b):
        pltpu.make_async_copy(t_hbm.at[pl.ds(0, GQ)], sbufs[b], ssems[b]).wait()

    def drain_d(b):
        pltpu.make_async_copy(t_hbm.at[pl.ds(0, GQ)], dbufs[b], dsems[b]).wait()

    for kk in range(3):
        fire(kk, kk)

    def group(ci, carry):
        for b in range(4):
            k = ci * 4 + b
            nxt = (b + 3) % 4

            def refire():
                drain_s(nxt)
                fire(k + 3, nxt)

            if b == 0:
                pl.when(ci > 0)(refire)
                pl.when(ci == 0)(lambda: fire(3, 3))
            else:
                pl.when(k + 3 < nstep)(refire)

            drain_s(b)
            drain_d(b)
            sbuf = sbufs[b]
            dbuf = dbufs[b]

            @plsc.parallel_loop(0, GQ, 1, unroll=2)
            def _(e):
                urow = sbuf.at[e]
                qrow = dbuf.at[e]
                for v in range(4):
                    urow[pl.ds(v * 16, 16)] = (
                        urow[pl.ds(v * 16, 16)] + qrow[pl.ds(v * 16, 16)]
                    )

            uoff = pl.multiple_of(base + k * GQ, 8)
            pltpu.async_copy(sbuf, u_hbm.at[pl.ds(uoff, GQ)], ssems[b])
        return carry

    lax.fori_loop(0, nstep // 4, group, 0)
    for b in range(4):
        drain_s(b)


def _make_gather():
    return pl.kernel(
        _gather_body,
        out_type=jax.ShapeDtypeStruct((E_PAD, 128), F32),
        mesh=_mesh(),
        scratch_types=[
            pltpu.VMEM((EC,), jnp.int32),
            pltpu.VMEM((EC,), jnp.int32),
        ]
        + [pltpu.VMEM((GQ, 128), F32) for _ in range(8)]
        + [pltpu.SemaphoreType.DMA for _ in range(8)],
    )


def _segmax_body(e_hbm, dst_hbm, sb_hbm, agg_hbm, acc_v, eb_v, db_v, sb_v):
    w = _wid()
    node_base = w * NPW
    pltpu.sync_copy(sb_hbm, sb_v)
    sbv = sb_v[pl.ds(w, 16)]
    e0 = sbv[0]
    e1 = sbv[1]
    e0a = (e0 // 8) * 8
    nch = (e1 - e0a + KD - 1) // KD
    negv = jnp.full((16,), NEG, F32)

    def chunk(ci, carry):
        off = e0a + ci * KD
        pltpu.sync_copy(e_hbm.at[pl.ds(off, KD)], eb_v)
        pltpu.sync_copy(dst_hbm.at[pl.ds(off, KD)], db_v.at[pl.ds(0, KD)])
        jlo = jnp.maximum(e0 - off, 0)
        jhi = jnp.minimum(e1 - off, KD)

        def edge(j, c2):
            cur = c2[0]
            r = c2[1:]
            d = db_v[pl.ds(j, 16)][0]
            isnew = d != cur

            @pl.when(isnew)
            def _():
                row = acc_v.at[cur - node_base]
                for v in range(8):
                    row[pl.ds(v * 16, 16)] = r[v]

            erow = eb_v.at[j]
            newr = []
            for v in range(8):
                ev = erow[pl.ds(v * 16, 16)]
                rv = jnp.where(isnew, negv, r[v])
                newr.append(jnp.maximum(rv, ev))
            return (d, *newr)

        return lax.fori_loop(jlo, jhi, edge, carry)

    init = (node_base, *([negv] * 8))
    fin = lax.fori_loop(0, nch, chunk, init)
    row = acc_v.at[fin[0] - node_base]
    for v in range(8):
        row[pl.ds(v * 16, 16)] = fin[1 + v]
    pltpu.sync_copy(acc_v, agg_hbm.at[pl.ds(node_base, NPW)])


def _make_segmax():
    return pl.kernel(
        _segmax_body,
        out_type=jax.ShapeDtypeStruct((NPAD, 128), F32),
        mesh=_mesh(),
        scratch_types=[
            pltpu.VMEM((NPW, 128), F32),
            pltpu.VMEM((KD, 128), F32),
            pltpu.VMEM((KD + 16,), jnp.int32),
            pltpu.VMEM((SB_LEN,), jnp.int32),
        ],
    )


def _relu(v):
    return jnp.maximum(v, 0.0)


def _node_pre_body(s_ref, x_ref, wh1, bh1, wh2, bh2, wh3p, bh3p, wp, ws, bf1,
                   t_ref, t2_ref):
    s = s_ref[...]
    h1 = _relu(s @ wh1[...] + bh1[...])
    h2 = _relu(h1 @ wh2[...] + bh2[...])
    dxp = h2 @ wh3p[...] + bh3p[...]
    wpv = wp[...]
    c = x_ref[...] @ wpv
    p = c + s @ ws[...] + bf1[...]
    nq = -(c + dxp @ wpv)
    t_ref[...] = jnp.concatenate([p, nq], axis=1)
    t2_ref[...] = jnp.concatenate([nq, p], axis=1)


def _edge_mlp_body(u_ref, wf2, bf2, wf3, bf3, e_ref):
    u = _relu(u_ref[...][:, :64])
    z = _relu(u @ wf2[...] + bf2[...])
    e_ref[...] = z @ wf3[...] + bf3[...]


def _update_body(s_ref, a_ref, wg1, bg1, wg2, bg2, wg3, bg3, o_ref):
    g1 = _relu(a_ref[...] @ wg1[...] + bg1[...])
    g2 = _relu(g1 @ wg2[...] + bg2[...])
    o_ref[...] = s_ref[...] + g2 @ wg3[...] + bg3[...]


def _inorm_blk(xx, w, b):
    y = xx @ w + b
    m = jnp.mean(y, axis=-1, keepdims=True)
    v = jnp.mean((y - m) ** 2, axis=-1, keepdims=True)
    return _relu((y - m) / jnp.sqrt(v + 1e-5))


def _heads_body(s_ref, wc1, bc1, wc2, bc2, wc3, bc3,
                wl10, bl10, wl20, bl20, wl30, bl30,
                wl11, bl11, wl21, bl21, wl31, bl31,
                wl12, bl12, wl22, bl22, wl32, bl32,
                cls_ref, reg_ref):
    st = s_ref[...]
    t = _inorm_blk(_inorm_blk(st, wc1[...], bc1[...]), wc2[...], bc2[...])
    cls_ref[...] = t @ wc3[...] + bc3[...]
    locs = [(wl10, bl10, wl20, bl20, wl30, bl30),
            (wl11, bl11, wl21, bl21, wl31, bl31),
            (wl12, bl12, wl22, bl22, wl32, bl32)]
    outs = []
    for (w1, b1, w2, b2, w3, b3) in locs:
        tl = _inorm_blk(_inorm_blk(st, w1[...], b1[...]), w2[...], b2[...])
        outs.append(tl @ w3[...] + b3[...])
    reg_ref[...] = jnp.concatenate(outs, axis=1)


def _full_spec(a):
    nd = a.ndim
    return pl.BlockSpec(a.shape, lambda i, _n=nd: (0,) * _n)


def _row_spec(rows, cols):
    return pl.BlockSpec((rows, cols), lambda i: (i, 0))


def _tc_call(body, grid, in_arrays, in_specs, out_shapes, out_specs):
    return pl.pallas_call(
        body,
        grid=grid,
        in_specs=in_specs,
        out_specs=out_specs,
        out_shape=out_shapes,
    )(*in_arrays)


def kernel(s, x, params, edge_index):
    ar = jnp.arange(N, dtype=jnp.int32)
    src = jnp.concatenate([edge_index[0].astype(jnp.int32), ar])
    dst = jnp.concatenate([edge_index[1].astype(jnp.int32), ar])
    order = jnp.argsort(dst)
    dst_s = dst[order]
    src_s = src[order]
    src_p = jnp.zeros((E_PAD,), jnp.int32).at[:E_TOT].set(src_s)
    dst_p = jnp.full((E_PAD,), N - 1, jnp.int32).at[:E_TOT].set(dst_s)
    sb = jnp.full((SB_LEN,), E_TOT, jnp.int32)
    sb = sb.at[: NW + 1].set(
        jnp.searchsorted(dst_s, jnp.arange(NW + 1, dtype=jnp.int32) * NPW).astype(
            jnp.int32
        )
    )

    s_p = jnp.zeros((NPAD, 128), F32).at[:N].set(s)
    x_p = jnp.zeros((NPAD, 128), F32).at[:N, :3].set(x)

    gather = _make_gather()
    segmax = _make_segmax()
    R = 1024
    RB = 2048

    for layer in params["layers"]:
        (wh1, bh1), (wh2, bh2), (wh3, bh3) = layer["h"]
        (wf1, bf1), (wf2, bf2), (wf3, bf3) = layer["f"]
        (wg1, bg1), (wg2, bg2), (wg3, bg3) = layer["g"]
        wp = jnp.zeros((128, 64), F32).at[:3].set(wf1[:3])
        ws = wf1[3:]
        wh3p = jnp.zeros((32, 128), F32).at[:, :3].set(wh3)
        bh3p = jnp.zeros((1, 128), F32).at[0, :3].set(bh3)

        pre_in = [s_p, x_p, wh1, bh1.reshape(1, -1), wh2, bh2.reshape(1, -1),
                  wh3p, bh3p, wp, ws, bf1.reshape(1, -1)]
        t_tab, t2_tab = _tc_call(
            _node_pre_body, (NPAD // R,), pre_in,
            [_row_spec(R, 128), _row_spec(R, 128)] + [_full_spec(a) for a in pre_in[2:]],
            [jax.ShapeDtypeStruct((NPAD, 128), F32)] * 2,
            [_row_spec(R, 128)] * 2,
        )

        u_raw = gather(t_tab, t2_tab, src_p, dst_p)

        mlp_in = [u_raw, wf2, bf2.reshape(1, -1), wf3, bf3.reshape(1, -1)]
        e2 = _tc_call(
            _edge_mlp_body, (E_PAD // RB,), mlp_in,
            [_row_spec(RB, 128)] + [_full_spec(a) for a in mlp_in[1:]],
            jax.ShapeDtypeStruct((E_PAD, 128), F32),
            _row_spec(RB, 128),
        )

        agg = segmax(e2, dst_p, sb)

        upd_in = [s_p, agg, wg1, bg1.reshape(1, -1), wg2, bg2.reshape(1, -1),
                  wg3, bg3.reshape(1, -1)]
        s_p = _tc_call(
            _update_body, (NPAD // R,), upd_in,
            [_row_spec(R, 128), _row_spec(R, 128)] + [_full_spec(a) for a in upd_in[2:]],
            jax.ShapeDtypeStruct((NPAD, 128), F32),
            _row_spec(R, 128),
        )

    (wc1, bc1), (wc2, bc2), (wc3, bc3) = params["mlp_class"]
    wc3p = jnp.zeros((128, 8), F32).at[:, :3].set(wc3)
    bc3p = jnp.zeros((1, 8), F32).at[0, :3].set(bc3)
    head_in = [s_p, wc1, bc1.reshape(1, -1), wc2, bc2.reshape(1, -1), wc3p, bc3p]
    for c in range(3):
        (w1, b1), (w2, b2), (w3, b3) = params["mlp_loc"][c]
        w3p = jnp.zeros((128, 8), F32).at[:, :7].set(w3)
        b3p = jnp.zeros((1, 8), F32).at[0, :7].set(b3)
        head_in += [w1, b1.reshape(1, -1), w2, b2.reshape(1, -1), w3p, b3p]
    cls8, reg24 = _tc_call(
        _heads_body, (NPAD // R,), head_in,
        [_row_spec(R, 128)] + [_full_spec(a) for a in head_in[1:]],
        [jax.ShapeDtypeStruct((NPAD, 8), F32), jax.ShapeDtypeStruct((NPAD, 24), F32)],
        [_row_spec(R, 8), _row_spec(R, 24)],
    )
    cls = cls8[:N, :3][None]
    reg = jnp.concatenate(
        [reg24[:N, 0:7], reg24[:N, 8:15], reg24[:N, 16:23]], axis=1
    )[None]
    return reg, cls


# speedup vs baseline: 1.1154x; 1.1154x over previous
import jax
import jax.numpy as jnp
from jax import lax
from jax.experimental import pallas as pl
from jax.experimental.pallas import tpu as pltpu
from jax.experimental.pallas import tpu_sc as plsc

N = 10000
NPAD = 10240
NW = 32
NPW = NPAD // NW
E_TOT = 330000
S = 1024
EC = 11264
E_PAD = NW * EC
KD = 512
NL = 264
GQ = 128
SB_LEN = 64
NC, NS = 2, 16
NEG = float("-inf")
F32 = jnp.float32


def _mesh():
    return plsc.VectorSubcoreMesh(core_axis_name="c", subcore_axis_name="s")


def _wid():
    return lax.axis_index("s") * NC + lax.axis_index("c")


def _gather_body(t_hbm, t2_hbm, src_hbm, dst_hbm, u_hbm, si_v, db_v,
                 ub0, ub1, ub2, ub3, slab_v, s0, s1, s2, s3):
    base = pl.multiple_of(_wid() * EC, 8)
    pltpu.sync_copy(src_hbm.at[pl.ds(base, EC)], si_v)
    pltpu.sync_copy(dst_hbm.at[pl.ds(base, EC)], db_v.at[pl.ds(0, EC)])
    bufs = (ub0, ub1, ub2, ub3)
    sems = (s0, s1, s2, s3)
    nstep = EC // GQ

    def fire(k, b):
        off = pl.multiple_of(k * GQ, 8)
        pltpu.async_copy(t_hbm.at[si_v.at[pl.ds(off, GQ)]], bufs[b], sems[b])

    def drain(b):
        pltpu.make_async_copy(t_hbm.at[pl.ds(0, GQ)], bufs[b], sems[b]).wait()

    for kk in range(3):
        fire(kk, kk)

    def group(ci, sbase):
        for b in range(4):
            k = ci * 4 + b
            nxt = (b + 3) % 4

            def refire():
                drain(nxt)
                fire(k + 3, nxt)

            if b == 0:
                pl.when(ci > 0)(refire)
                pl.when(ci == 0)(lambda: fire(3, 3))
            else:
                pl.when(k + 3 < nstep)(refire)

            drain(b)
            dfirst = db_v[pl.ds(k * GQ, 16)][0]
            dlast = db_v[pl.ds(k * GQ + GQ - 1, 16)][0]
            reload = (dlast - sbase) >= NL
            nb = pl.multiple_of((dfirst // 8) * 8, 8)

            @pl.when(reload)
            def _():
                pltpu.sync_copy(t2_hbm.at[pl.ds(nb, NL)], slab_v)

            sbase = jnp.where(reload, nb, sbase)
            sb = sbase
            buf = bufs[b]
            koff = k * GQ

            @plsc.parallel_loop(0, GQ, 1, unroll=2)
            def _(e):
                d = db_v[pl.ds(koff + e, 16)][0]
                urow = buf.at[e]
                qrow = slab_v.at[d - sb]
                for v in range(4):
                    urow[pl.ds(v * 16, 16)] = (
                        urow[pl.ds(v * 16, 16)] + qrow[pl.ds(v * 16, 16)]
                    )

            uoff = pl.multiple_of(base + k * GQ, 8)
            pltpu.async_copy(buf, u_hbm.at[pl.ds(uoff, GQ)], sems[b])
        return sbase

    lax.fori_loop(0, nstep // 4, group, jnp.int32(-2 * NL))
    for b in range(4):
        drain(b)


def _make_gather():
    return pl.kernel(
        _gather_body,
        out_type=jax.ShapeDtypeStruct((E_PAD, 128), F32),
        mesh=_mesh(),
        scratch_types=[
            pltpu.VMEM((EC,), jnp.int32),
            pltpu.VMEM((EC + 16,), jnp.int32),
            pltpu.VMEM((GQ, 128), F32),
            pltpu.VMEM((GQ, 128), F32),
            pltpu.VMEM((GQ, 128), F32),
            pltpu.VMEM((GQ, 128), F32),
            pltpu.VMEM((NL, 128), F32),
            pltpu.SemaphoreType.DMA,
            pltpu.SemaphoreType.DMA,
            pltpu.SemaphoreType.DMA,
            pltpu.SemaphoreType.DMA,
        ],
    )


def _segmax_body(e_hbm, dst_hbm, sb_hbm, agg_hbm, acc_v, eb_v, db_v, sb_v):
    w = _wid()
    node_base = w * NPW
    pltpu.sync_copy(sb_hbm, sb_v)
    sbv = sb_v[pl.ds(w, 16)]
    e0 = sbv[0]
    e1 = sbv[1]
    e0a = (e0 // 8) * 8
    nch = (e1 - e0a + KD - 1) // KD
    negv = jnp.full((16,), NEG, F32)

    def chunk(ci, carry):
        off = e0a + ci * KD
        pltpu.sync_copy(e_hbm.at[pl.ds(off, KD)], eb_v)
        pltpu.sync_copy(dst_hbm.at[pl.ds(off, KD)], db_v.at[pl.ds(0, KD)])
        jlo = jnp.maximum(e0 - off, 0)
        jhi = jnp.minimum(e1 - off, KD)

        def edge(j, c2):
            cur = c2[0]
            r = c2[1:]
            d = db_v[pl.ds(j, 16)][0]
            isnew = d != cur

            @pl.when(isnew)
            def _():
                row = acc_v.at[cur - node_base]
                for v in range(8):
                    row[pl.ds(v * 16, 16)] = r[v]

            erow = eb_v.at[j]
            newr = []
            for v in range(8):
                ev = erow[pl.ds(v * 16, 16)]
                rv = jnp.where(isnew, negv, r[v])
                newr.append(jnp.maximum(rv, ev))
            return (d, *newr)

        return lax.fori_loop(jlo, jhi, edge, carry)

    init = (node_base, *([negv] * 8))
    fin = lax.fori_loop(0, nch, chunk, init)
    row = acc_v.at[fin[0] - node_base]
    for v in range(8):
        row[pl.ds(v * 16, 16)] = fin[1 + v]
    pltpu.sync_copy(acc_v, agg_hbm.at[pl.ds(node_base, NPW)])


def _make_segmax():
    return pl.kernel(
        _segmax_body,
        out_type=jax.ShapeDtypeStruct((NPAD, 128), F32),
        mesh=_mesh(),
        scratch_types=[
            pltpu.VMEM((NPW, 128), F32),
            pltpu.VMEM((KD, 128), F32),
            pltpu.VMEM((KD + 16,), jnp.int32),
            pltpu.VMEM((SB_LEN,), jnp.int32),
        ],
    )


def _relu(v):
    return jnp.maximum(v, 0.0)


def _node_pre_body(s_ref, x_ref, wh1, bh1, wh2, bh2, wh3p, bh3p, wp, ws, bf1,
                   t_ref, t2_ref):
    s = s_ref[...]
    h1 = _relu(s @ wh1[...] + bh1[...])
    h2 = _relu(h1 @ wh2[...] + bh2[...])
    dxp = h2 @ wh3p[...] + bh3p[...]
    wpv = wp[...]
    c = x_ref[...] @ wpv
    p = c + s @ ws[...] + bf1[...]
    nq = -(c + dxp @ wpv)
    t_ref[...] = jnp.concatenate([p, nq], axis=1)
    t2_ref[...] = jnp.concatenate([nq, p], axis=1)


def _edge_mlp_body(u_ref, wf2, bf2, wf3, bf3, e_ref):
    u = _relu(u_ref[...][:, :64])
    z = _relu(u @ wf2[...] + bf2[...])
    e_ref[...] = z @ wf3[...] + bf3[...]


def _update_body(s_ref, a_ref, wg1, bg1, wg2, bg2, wg3, bg3, o_ref):
    g1 = _relu(a_ref[...] @ wg1[...] + bg1[...])
    g2 = _relu(g1 @ wg2[...] + bg2[...])
    o_ref[...] = s_ref[...] + g2 @ wg3[...] + bg3[...]


def _inorm_blk(xx, w, b):
    y = xx @ w + b
    m = jnp.mean(y, axis=-1, keepdims=True)
    v = jnp.mean((y - m) ** 2, axis=-1, keepdims=True)
    return _relu((y - m) / jnp.sqrt(v + 1e-5))


def _heads_body(s_ref, wc1, bc1, wc2, bc2, wc3, bc3,
                wl10, bl10, wl20, bl20, wl30, bl30,
                wl11, bl11, wl21, bl21, wl31, bl31,
                wl12, bl12, wl22, bl22, wl32, bl32,
                cls_ref, reg_ref):
    st = s_ref[...]
    t = _inorm_blk(_inorm_blk(st, wc1[...], bc1[...]), wc2[...], bc2[...])
    cls_ref[...] = t @ wc3[...] + bc3[...]
    locs = [(wl10, bl10, wl20, bl20, wl30, bl30),
            (wl11, bl11, wl21, bl21, wl31, bl31),
            (wl12, bl12, wl22, bl22, wl32, bl32)]
    outs = []
    for (w1, b1, w2, b2, w3, b3) in locs:
        tl = _inorm_blk(_inorm_blk(st, w1[...], b1[...]), w2[...], b2[...])
        outs.append(tl @ w3[...] + b3[...])
    reg_ref[...] = jnp.concatenate(outs, axis=1)


def _full_spec(a):
    nd = a.ndim
    return pl.BlockSpec(a.shape, lambda i, _n=nd: (0,) * _n)


def _row_spec(rows, cols):
    return pl.BlockSpec((rows, cols), lambda i: (i, 0))


def _tc_call(body, grid, in_arrays, in_specs, out_shapes, out_specs):
    return pl.pallas_call(
        body,
        grid=grid,
        in_specs=in_specs,
        out_specs=out_specs,
        out_shape=out_shapes,
    )(*in_arrays)


def kernel(s, x, params, edge_index):
    ar = jnp.arange(N, dtype=jnp.int32)
    src = jnp.concatenate([edge_index[0].astype(jnp.int32), ar])
    dst = jnp.concatenate([edge_index[1].astype(jnp.int32), ar])
    key = jnp.sort(dst * 16384 + src)
    dst_s = key >> 14
    src_s = key & 16383
    src_p = jnp.zeros((E_PAD,), jnp.int32).at[:E_TOT].set(src_s)
    dst_p = jnp.full((E_PAD,), N - 1, jnp.int32).at[:E_TOT].set(dst_s)
    sb = jnp.full((SB_LEN,), E_TOT, jnp.int32)
    sb = sb.at[: NW + 1].set(
        jnp.searchsorted(dst_s, jnp.arange(NW + 1, dtype=jnp.int32) * NPW).astype(
            jnp.int32
        )
    )

    s_p = jnp.zeros((NPAD, 128), F32).at[:N].set(s)
    x_p = jnp.zeros((NPAD, 128), F32).at[:N, :3].set(x)

    gather = _make_gather()
    segmax = _make_segmax()
    R = 1024
    RB = 2048

    for layer in params["layers"]:
        (wh1, bh1), (wh2, bh2), (wh3, bh3) = layer["h"]
        (wf1, bf1), (wf2, bf2), (wf3, bf3) = layer["f"]
        (wg1, bg1), (wg2, bg2), (wg3, bg3) = layer["g"]
        wp = jnp.zeros((128, 64), F32).at[:3].set(wf1[:3])
        ws = wf1[3:]
        wh3p = jnp.zeros((32, 128), F32).at[:, :3].set(wh3)
        bh3p = jnp.zeros((1, 128), F32).at[0, :3].set(bh3)

        pre_in = [s_p, x_p, wh1, bh1.reshape(1, -1), wh2, bh2.reshape(1, -1),
                  wh3p, bh3p, wp, ws, bf1.reshape(1, -1)]
        t_tab, t2_tab = _tc_call(
            _node_pre_body, (NPAD // R,), pre_in,
            [_row_spec(R, 128), _row_spec(R, 128)] + [_full_spec(a) for a in pre_in[2:]],
            [jax.ShapeDtypeStruct((NPAD, 128), F32)] * 2,
            [_row_spec(R, 128)] * 2,
        )

        u_raw = gather(t_tab, t2_tab, src_p, dst_p)

        mlp_in = [u_raw, wf2, bf2.reshape(1, -1), wf3, bf3.reshape(1, -1)]
        e2 = _tc_call(
            _edge_mlp_body, (E_PAD // RB,), mlp_in,
            [_row_spec(RB, 128)] + [_full_spec(a) for a in mlp_in[1:]],
            jax.ShapeDtypeStruct((E_PAD, 128), F32),
            _row_spec(RB, 128),
        )

        agg = segmax(e2, dst_p, sb)

        upd_in = [s_p, agg, wg1, bg1.reshape(1, -1), wg2, bg2.reshape(1, -1),
                  wg3, bg3.reshape(1, -1)]
        s_p = _tc_call(
            _update_body, (NPAD // R,), upd_in,
            [_row_spec(R, 128), _row_spec(R, 128)] + [_full_spec(a) for a in upd_in[2:]],
            jax.ShapeDtypeStruct((NPAD, 128), F32),
            _row_spec(R, 128),
        )

    (wc1, bc1), (wc2, bc2), (wc3, bc3) = params["mlp_class"]
    wc3p = jnp.zeros((128, 8), F32).at[:, :3].set(wc3)
    bc3p = jnp.zeros((1, 8), F32).at[0, :3].set(bc3)
    head_in = [s_p, wc1, bc1.reshape(1, -1), wc2, bc2.reshape(1, -1), wc3p, bc3p]
    for c in range(3):
        (w1, b1), (w2, b2), (w3, b3) = params["mlp_loc"][c]
        w3p = jnp.zeros((128, 8), F32).at[:, :7].set(w3)
        b3p = jnp.zeros((1, 8), F32).at[0, :7].set(b3)
        head_in += [w1, b1.reshape(1, -1), w2, b2.reshape(1, -1), w3p, b3p]
    cls8, reg24 = _tc_call(
        _heads_body, (NPAD // R,), head_in,
        [_row_spec(R, 128)] + [_full_spec(a) for a in head_in[1:]],
        [jax.ShapeDtypeStruct((NPAD, 8), F32), jax.ShapeDtypeStruct((NPAD, 24), F32)],
        [_row_spec(R, 8), _row_spec(R, 24)],
    )
    cls = cls8[:N, :3][None]
    reg = jnp.concatenate(
        [reg24[:N, 0:7], reg24[:N, 8:15], reg24[:N, 16:23]], axis=1
    )[None]
    return reg, cls
